# baseline (speedup 1.0000x reference)
import jax
import jax.numpy as jnp
from jax.experimental import pallas as pl
from jax.experimental.pallas import tpu as pltpu

_BATCH = 16384
_EMBED = 256
_COMMIT = 0.5
_BT = 2048
_NSUB = 2
_SB = _BT // _NSUB


def _ln(x, g, b):
    mu = jnp.mean(x, axis=-1, keepdims=True)
    var = jnp.mean((x - mu) ** 2, axis=-1, keepdims=True)
    return (x - mu) / jnp.sqrt(var + 1e-5) * g + b


def _rqvae_kernel(x_ref,
                  eW0, eb0, eg0, ebt0, eW1, eb1, eg1, ebt1, eW2, eb2, eg2, ebt2,
                  dW0, db0, dW1, db1, dW2, db2, dg0, dbt0, dg1, dbt1,
                  cb0_ref, cb1_ref, cb2_ref,
                  rec_ref, loss_ref, c0_ref, c1_ref, c2_ref,
                  s_wide, s_mid, s_cn):
    @pl.when(pl.program_id(0) == 0)
    def _():
        for j, cbr in enumerate((cb0_ref, cb1_ref, cb2_ref)):
            cbv = cbr[...]
            s_cn[j:j + 1, :cbv.shape[0]] = jnp.sum(cbv * cbv, axis=1)[None, :]

    loss_acc = jnp.float32(0.0)
    for h in range(_NSUB):
        rows = pl.ds(h * _SB, _SB)
        s_wide[rows, :] = jax.nn.relu(_ln(
            jnp.dot(x_ref[rows, :], eW0[...], preferred_element_type=jnp.float32)
            + eb0[...], eg0[...], ebt0[...]))
        s_mid[rows, :] = jax.nn.relu(_ln(
            jnp.dot(s_wide[rows, :], eW1[...], preferred_element_type=jnp.float32)
            + eb1[...], eg1[...], ebt1[...]))
        z = _ln(jnp.dot(s_mid[rows, :], eW2[...], preferred_element_type=jnp.float32)
                + eb2[...], eg2[...], ebt2[...])

        residual = z
        qsum = jnp.zeros_like(z)
        for j, (cb_ref, c_ref) in enumerate(
                ((cb0_ref, c0_ref), (cb1_ref, c1_ref), (cb2_ref, c2_ref))):
            cb = cb_ref[...]
            k = cb.shape[0]
            scores = jax.lax.dot_general(residual, cb, (((1,), (1,)), ((), ())),
                                         preferred_element_type=jnp.float32)
            rn = jnp.sum(residual * residual, axis=1, keepdims=True)
            d2 = (rn + s_cn[j:j + 1, :k]) - 2.0 * scores
            dmin = jnp.min(d2, axis=1, keepdims=True)
            iota = jax.lax.broadcasted_iota(jnp.int32, d2.shape, 1)
            idx = jnp.min(jnp.where(d2 <= dmin, iota, k), axis=1, keepdims=True)
            onehot = (iota == idx).astype(jnp.float32)
            cb_1 = cb.astype(jnp.bfloat16).astype(jnp.float32)
            cb_r = cb - cb_1
            cb_2 = cb_r.astype(jnp.bfloat16).astype(jnp.float32)
            cb_3 = cb_r - cb_2
            zq = ((jnp.dot(onehot, cb_1, preferred_element_type=jnp.float32)
                   + jnp.dot(onehot, cb_2, preferred_element_type=jnp.float32))
                  + jnp.dot(onehot, cb_3, preferred_element_type=jnp.float32))
            loss_acc += jnp.sum((zq - residual) ** 2)
            qsum = qsum + zq
            residual = residual - zq
            c_ref[rows, :] = idx

        s_mid[rows, :] = jax.nn.relu(_ln(
            jnp.dot(qsum, dW0[...], preferred_element_type=jnp.float32)
            + db0[...], dg0[...], dbt0[...]))
        s_wide[rows, :] = jax.nn.relu(_ln(
            jnp.dot(s_mid[rows, :], dW1[...], preferred_element_type=jnp.float32)
            + db1[...], dg1[...], dbt1[...]))
        rec_ref[rows, :] = jnp.dot(s_wide[rows, :], dW2[...],
                                   preferred_element_type=jnp.float32) + db2[...]

    @pl.when(pl.program_id(0) == 0)
    def _():
        loss_ref[...] = jnp.zeros((1, 1), jnp.float32)
    loss_ref[...] += jnp.reshape(loss_acc * (_COMMIT / (_BATCH * _EMBED)), (1, 1))


def kernel(x, enc_W0, enc_b0, enc_g0, enc_beta0, enc_W1, enc_b1, enc_g1, enc_beta1,
           enc_W2, enc_b2, enc_g2, enc_beta2,
           dec_W0, dec_b0, dec_W1, dec_b1, dec_W2, dec_b2,
           dec_g0, dec_beta0, dec_g1, dec_beta1,
           cb0, cb1, cb2):
    n_steps = _BATCH // _BT

    def _full(a):
        return pl.BlockSpec(a.shape, lambda i: (0,) * a.ndim)

    in_specs = [pl.BlockSpec((_BT, x.shape[1]), lambda i: (i, 0))]
    weights = (enc_W0, enc_b0, enc_g0, enc_beta0, enc_W1, enc_b1, enc_g1, enc_beta1,
               enc_W2, enc_b2, enc_g2, enc_beta2,
               dec_W0, dec_b0, dec_W1, dec_b1, dec_W2, dec_b2,
               dec_g0, dec_beta0, dec_g1, dec_beta1, cb0, cb1, cb2)
    in_specs += [_full(w) for w in weights]

    out_shapes = (
        jax.ShapeDtypeStruct((_BATCH, x.shape[1]), jnp.float32),
        jax.ShapeDtypeStruct((1, 1), jnp.float32),
        jax.ShapeDtypeStruct((_BATCH, 1), jnp.int32),
        jax.ShapeDtypeStruct((_BATCH, 1), jnp.int32),
        jax.ShapeDtypeStruct((_BATCH, 1), jnp.int32),
    )
    out_specs = (
        pl.BlockSpec((_BT, x.shape[1]), lambda i: (i, 0)),
        pl.BlockSpec((1, 1), lambda i: (0, 0)),
        pl.BlockSpec((_BT, 1), lambda i: (i, 0)),
        pl.BlockSpec((_BT, 1), lambda i: (i, 0)),
        pl.BlockSpec((_BT, 1), lambda i: (i, 0)),
    )

    rec, loss, c0, c1, c2 = pl.pallas_call(
        _rqvae_kernel,
        grid=(n_steps,),
        in_specs=in_specs,
        out_specs=out_specs,
        out_shape=out_shapes,
        scratch_shapes=[
            pltpu.VMEM((_BT, 768), jnp.float32),
            pltpu.VMEM((_BT, 512), jnp.float32),
            pltpu.VMEM((8, 512), jnp.float32),
        ],
        compiler_params=pltpu.CompilerParams(
            dimension_semantics=("arbitrary",),
        ),
    )(x, *weights)

    codes = jnp.concatenate([c0, c1, c2], axis=1)
    return (rec, loss[0, 0], codes)


# speedup vs baseline: 1.0109x; 1.0109x over previous
import jax
import jax.numpy as jnp
from jax.experimental import pallas as pl
from jax.experimental.pallas import tpu as pltpu

_BATCH = 16384
_EMBED = 256
_COMMIT = 0.5
_BT = 2048
_NSUB = 1
_SB = _BT // _NSUB


def _ln(x, g, b):
    mu = jnp.mean(x, axis=-1, keepdims=True)
    var = jnp.mean((x - mu) ** 2, axis=-1, keepdims=True)
    return (x - mu) / jnp.sqrt(var + 1e-5) * g + b


def _rqvae_kernel(x_ref,
                  eW0, eb0, eg0, ebt0, eW1, eb1, eg1, ebt1, eW2, eb2, eg2, ebt2,
                  dW0, db0, dW1, db1, dW2, db2, dg0, dbt0, dg1, dbt1,
                  cb0_ref, cb1_ref, cb2_ref,
                  rec_ref, loss_ref, c0_ref, c1_ref, c2_ref,
                  s_wide, s_mid, s_cn):
    @pl.when(pl.program_id(0) == 0)
    def _():
        for j, cbr in enumerate((cb0_ref, cb1_ref, cb2_ref)):
            cbv = cbr[...]
            s_cn[j:j + 1, :cbv.shape[0]] = jnp.sum(cbv * cbv, axis=1)[None, :]

    loss_acc = jnp.float32(0.0)
    for h in range(_NSUB):
        rows = pl.ds(h * _SB, _SB)
        s_wide[rows, :] = jax.nn.relu(_ln(
            jnp.dot(x_ref[rows, :], eW0[...], preferred_element_type=jnp.float32)
            + eb0[...], eg0[...], ebt0[...]))
        s_mid[rows, :] = jax.nn.relu(_ln(
            jnp.dot(s_wide[rows, :], eW1[...], preferred_element_type=jnp.float32)
            + eb1[...], eg1[...], ebt1[...]))
        z = _ln(jnp.dot(s_mid[rows, :], eW2[...], preferred_element_type=jnp.float32)
                + eb2[...], eg2[...], ebt2[...])

        residual = z
        qsum = jnp.zeros_like(z)
        for j, (cb_ref, c_ref) in enumerate(
                ((cb0_ref, c0_ref), (cb1_ref, c1_ref), (cb2_ref, c2_ref))):
            cb = cb_ref[...]
            k = cb.shape[0]
            scores = jax.lax.dot_general(residual, cb, (((1,), (1,)), ((), ())),
                                         preferred_element_type=jnp.float32)
            rn = jnp.sum(residual * residual, axis=1, keepdims=True)
            d2 = (rn + s_cn[j:j + 1, :k]) - 2.0 * scores
            dmin = jnp.min(d2, axis=1, keepdims=True)
            iota = jax.lax.broadcasted_iota(jnp.int32, d2.shape, 1)
            idx = jnp.min(jnp.where(d2 <= dmin, iota, k), axis=1, keepdims=True)
            onehot = (iota == idx).astype(jnp.float32)
            cb_1 = cb.astype(jnp.bfloat16).astype(jnp.float32)
            cb_r = cb - cb_1
            cb_2 = cb_r.astype(jnp.bfloat16).astype(jnp.float32)
            cb_3 = cb_r - cb_2
            zq = ((jnp.dot(onehot, cb_1, preferred_element_type=jnp.float32)
                   + jnp.dot(onehot, cb_2, preferred_element_type=jnp.float32))
                  + jnp.dot(onehot, cb_3, preferred_element_type=jnp.float32))
            loss_acc += jnp.sum((zq - residual) ** 2)
            qsum = qsum + zq
            residual = residual - zq
            c_ref[rows, :] = idx

        s_mid[rows, :] = jax.nn.relu(_ln(
            jnp.dot(qsum, dW0[...], preferred_element_type=jnp.float32)
            + db0[...], dg0[...], dbt0[...]))
        s_wide[rows, :] = jax.nn.relu(_ln(
            jnp.dot(s_mid[rows, :], dW1[...], preferred_element_type=jnp.float32)
            + db1[...], dg1[...], dbt1[...]))
        rec_ref[rows, :] = jnp.dot(s_wide[rows, :], dW2[...],
                                   preferred_element_type=jnp.float32) + db2[...]

    @pl.when(pl.program_id(0) == 0)
    def _():
        loss_ref[...] = jnp.zeros((1, 1), jnp.float32)
    loss_ref[...] += jnp.reshape(loss_acc * (_COMMIT / (_BATCH * _EMBED)), (1, 1))


def kernel(x, enc_W0, enc_b0, enc_g0, enc_beta0, enc_W1, enc_b1, enc_g1, enc_beta1,
           enc_W2, enc_b2, enc_g2, enc_beta2,
           dec_W0, dec_b0, dec_W1, dec_b1, dec_W2, dec_b2,
           dec_g0, dec_beta0, dec_g1, dec_beta1,
           cb0, cb1, cb2):
    n_steps = _BATCH // _BT

    def _full(a):
        return pl.BlockSpec(a.shape, lambda i: (0,) * a.ndim)

    in_specs = [pl.BlockSpec((_BT, x.shape[1]), lambda i: (i, 0))]
    weights = (enc_W0, enc_b0, enc_g0, enc_beta0, enc_W1, enc_b1, enc_g1, enc_beta1,
               enc_W2, enc_b2, enc_g2, enc_beta2,
               dec_W0, dec_b0, dec_W1, dec_b1, dec_W2, dec_b2,
               dec_g0, dec_beta0, dec_g1, dec_beta1, cb0, cb1, cb2)
    in_specs += [_full(w) for w in weights]

    out_shapes = (
        jax.ShapeDtypeStruct((_BATCH, x.shape[1]), jnp.float32),
        jax.ShapeDtypeStruct((1, 1), jnp.float32),
        jax.ShapeDtypeStruct((_BATCH, 1), jnp.int32),
        jax.ShapeDtypeStruct((_BATCH, 1), jnp.int32),
        jax.ShapeDtypeStruct((_BATCH, 1), jnp.int32),
    )
    out_specs = (
        pl.BlockSpec((_BT, x.shape[1]), lambda i: (i, 0)),
        pl.BlockSpec((1, 1), lambda i: (0, 0)),
        pl.BlockSpec((_BT, 1), lambda i: (i, 0)),
        pl.BlockSpec((_BT, 1), lambda i: (i, 0)),
        pl.BlockSpec((_BT, 1), lambda i: (i, 0)),
    )

    rec, loss, c0, c1, c2 = pl.pallas_call(
        _rqvae_kernel,
        grid=(n_steps,),
        in_specs=in_specs,
        out_specs=out_specs,
        out_shape=out_shapes,
        scratch_shapes=[
            pltpu.VMEM((_BT, 768), jnp.float32),
            pltpu.VMEM((_BT, 512), jnp.float32),
            pltpu.VMEM((8, 512), jnp.float32),
        ],
        compiler_params=pltpu.CompilerParams(
            dimension_semantics=("arbitrary",),
        ),
    )(x, *weights)

    codes = jnp.concatenate([c0, c1, c2], axis=1)
    return (rec, loss[0, 0], codes)


# speedup vs baseline: 1.0180x; 1.0070x over previous
import jax
import jax.numpy as jnp
from jax.experimental import pallas as pl
from jax.experimental.pallas import tpu as pltpu

_BATCH = 16384
_EMBED = 256
_COMMIT = 0.5
_BT = 2048
_NSUB = 1
_SB = _BT // _NSUB


def _ln(x, g, b):
    mu = jnp.mean(x, axis=-1, keepdims=True)
    var = jnp.mean((x - mu) ** 2, axis=-1, keepdims=True)
    return (x - mu) / jnp.sqrt(var + 1e-5) * g + b


def _rqvae_kernel(x_ref,
                  eW0, eb0, eg0, ebt0, eW1, eb1, eg1, ebt1, eW2, eb2, eg2, ebt2,
                  dW0, db0, dW1, db1, dW2, db2, dg0, dbt0, dg1, dbt1,
                  cb0_ref, cb1_ref, cb2_ref,
                  rec_ref, loss_ref, c0_ref, c1_ref, c2_ref,
                  s_wide, s_mid, s_cn):
    @pl.when(pl.program_id(0) == 0)
    def _():
        for j, cbr in enumerate((cb0_ref, cb1_ref, cb2_ref)):
            cbv = cbr[...]
            s_cn[j:j + 1, :cbv.shape[0]] = jnp.sum(cbv * cbv, axis=1)[None, :]

    loss_acc = jnp.float32(0.0)
    for h in range(_NSUB):
        rows = pl.ds(h * _SB, _SB)
        s_wide[rows, :] = jax.nn.relu(_ln(
            jnp.dot(x_ref[rows, :], eW0[...], preferred_element_type=jnp.float32)
            + eb0[...], eg0[...], ebt0[...]))
        s_mid[rows, :] = jax.nn.relu(_ln(
            jnp.dot(s_wide[rows, :], eW1[...], preferred_element_type=jnp.float32)
            + eb1[...], eg1[...], ebt1[...]))
        z = _ln(jnp.dot(s_mid[rows, :], eW2[...], preferred_element_type=jnp.float32)
                + eb2[...], eg2[...], ebt2[...])

        residual = z
        qsum = jnp.zeros_like(z)
        for j, (cb_ref, c_ref) in enumerate(
                ((cb0_ref, c0_ref), (cb1_ref, c1_ref), (cb2_ref, c2_ref))):
            cb = cb_ref[...]
            k = cb.shape[0]
            scores = jax.lax.dot_general(residual, cb, (((1,), (1,)), ((), ())),
                                         preferred_element_type=jnp.float32)
            rn = jnp.sum(residual * residual, axis=1, keepdims=True)
            d2 = (rn + s_cn[j:j + 1, :k]) - 2.0 * scores
            dmin = jnp.min(d2, axis=1, keepdims=True)
            iota = jax.lax.broadcasted_iota(jnp.int32, d2.shape, 1)
            idx = jnp.min(jnp.where(d2 <= dmin, iota, k), axis=1, keepdims=True)
            onehot = (iota == idx).astype(jnp.float32)
            cb_1 = cb.astype(jnp.bfloat16).astype(jnp.float32)
            cb_r = cb - cb_1
            cb_2 = cb_r.astype(jnp.bfloat16).astype(jnp.float32)
            cb_3 = cb_r - cb_2
            zq = ((jnp.dot(onehot, cb_1, preferred_element_type=jnp.float32)
                   + jnp.dot(onehot, cb_2, preferred_element_type=jnp.float32))
                  + jnp.dot(onehot, cb_3, preferred_element_type=jnp.float32))
            loss_acc += jnp.sum(dmin)
            qsum = qsum + zq
            if j < 2:
                residual = residual - zq
            c_ref[rows, :] = idx

        s_mid[rows, :] = jax.nn.relu(_ln(
            jnp.dot(qsum, dW0[...], preferred_element_type=jnp.float32)
            + db0[...], dg0[...], dbt0[...]))
        s_wide[rows, :] = jax.nn.relu(_ln(
            jnp.dot(s_mid[rows, :], dW1[...], preferred_element_type=jnp.float32)
            + db1[...], dg1[...], dbt1[...]))
        rec_ref[rows, :] = jnp.dot(s_wide[rows, :], dW2[...],
                                   preferred_element_type=jnp.float32) + db2[...]

    @pl.when(pl.program_id(0) == 0)
    def _():
        loss_ref[...] = jnp.zeros((1, 1), jnp.float32)
    loss_ref[...] += jnp.reshape(loss_acc * (_COMMIT / (_BATCH * _EMBED)), (1, 1))


def kernel(x, enc_W0, enc_b0, enc_g0, enc_beta0, enc_W1, enc_b1, enc_g1, enc_beta1,
           enc_W2, enc_b2, enc_g2, enc_beta2,
           dec_W0, dec_b0, dec_W1, dec_b1, dec_W2, dec_b2,
           dec_g0, dec_beta0, dec_g1, dec_beta1,
           cb0, cb1, cb2):
    n_steps = _BATCH // _BT

    def _full(a):
        return pl.BlockSpec(a.shape, lambda i: (0,) * a.ndim)

    in_specs = [pl.BlockSpec((_BT, x.shape[1]), lambda i: (i, 0))]
    weights = (enc_W0, enc_b0, enc_g0, enc_beta0, enc_W1, enc_b1, enc_g1, enc_beta1,
               enc_W2, enc_b2, enc_g2, enc_beta2,
               dec_W0, dec_b0, dec_W1, dec_b1, dec_W2, dec_b2,
               dec_g0, dec_beta0, dec_g1, dec_beta1, cb0, cb1, cb2)
    in_specs += [_full(w) for w in weights]

    out_shapes = (
        jax.ShapeDtypeStruct((_BATCH, x.shape[1]), jnp.float32),
        jax.ShapeDtypeStruct((1, 1), jnp.float32),
        jax.ShapeDtypeStruct((_BATCH, 1), jnp.int32),
        jax.ShapeDtypeStruct((_BATCH, 1), jnp.int32),
        jax.ShapeDtypeStruct((_BATCH, 1), jnp.int32),
    )
    out_specs = (
        pl.BlockSpec((_BT, x.shape[1]), lambda i: (i, 0)),
        pl.BlockSpec((1, 1), lambda i: (0, 0)),
        pl.BlockSpec((_BT, 1), lambda i: (i, 0)),
        pl.BlockSpec((_BT, 1), lambda i: (i, 0)),
        pl.BlockSpec((_BT, 1), lambda i: (i, 0)),
    )

    rec, loss, c0, c1, c2 = pl.pallas_call(
        _rqvae_kernel,
        grid=(n_steps,),
        in_specs=in_specs,
        out_specs=out_specs,
        out_shape=out_shapes,
        scratch_shapes=[
            pltpu.VMEM((_BT, 768), jnp.float32),
            pltpu.VMEM((_BT, 512), jnp.float32),
            pltpu.VMEM((8, 512), jnp.float32),
        ],
        compiler_params=pltpu.CompilerParams(
            dimension_semantics=("arbitrary",),
        ),
    )(x, *weights)

    codes = jnp.concatenate([c0, c1, c2], axis=1)
    return (rec, loss[0, 0], codes)
